# baseline (speedup 1.0000x reference)
import functools

import jax
import jax.numpy as jnp
from jax import lax
from jax.experimental import pallas as pl
from jax.experimental.pallas import tpu as pltpu
from jax.experimental.pallas import tpu_sc as plsc

B = 4096
U = 4096
I = 8192
D = 64

NC = 2
NS = 16
NW = NC * NS
L = 16
BPW = B // NW
NG = BPW // L
DC = D // L

_mesh = plsc.VectorSubcoreMesh(core_axis_name="c", subcore_axis_name="s")


@functools.partial(
    pl.kernel,
    mesh=_mesh,
    out_type=(
        jax.ShapeDtypeStruct((B,), jnp.float32),
        jax.ShapeDtypeStruct((B,), jnp.float32),
    ),
    scratch_types=dict(
        idx_u=pltpu.VMEM((BPW,), jnp.int32),
        idx_i=pltpu.VMEM((BPW,), jnp.int32),
        idx_j=pltpu.VMEM((BPW,), jnp.int32),
        idx_il=pltpu.VMEM((BPW,), jnp.int32),
        idx_jl=pltpu.VMEM((BPW,), jnp.int32),
        u_rows=pltpu.VMEM((BPW, D), jnp.float32),
        ei_rows=pltpu.VMEM((BPW, D), jnp.float32),
        ej_rows=pltpu.VMEM((BPW, D), jnp.float32),
        lin_u=pltpu.VMEM((BPW,), jnp.float32),
        lin_iv=pltpu.VMEM((BPW,), jnp.float32),
        lin_jv=pltpu.VMEM((BPW,), jnp.float32),
        out_i_v=pltpu.VMEM((BPW,), jnp.float32),
        out_j_v=pltpu.VMEM((BPW,), jnp.float32),
        sem_r=pltpu.SemaphoreType.DMA,
        sem_l=pltpu.SemaphoreType.DMA,
    ),
    compiler_params=pltpu.CompilerParams(
        needs_layout_passes=False, use_tc_tiling_on_sc=True),
)
def _fm_kernel(
    user_hbm, item_i_hbm, item_j_hbm, lin_hbm, eu_hbm, eit_hbm,
    out_i_hbm, out_j_hbm,
    *, idx_u, idx_i, idx_j, idx_il, idx_jl,
    u_rows, ei_rows, ej_rows, lin_u, lin_iv, lin_jv, out_i_v, out_j_v,
    sem_r, sem_l,
):
    wid = lax.axis_index("s") * NC + lax.axis_index("c")
    base = wid * BPW

    with jax.named_scope("stage"):
        stage = [
            pltpu.async_copy(user_hbm.at[pl.ds(base, BPW)], idx_u, sem_l),
            pltpu.async_copy(item_i_hbm.at[pl.ds(base, BPW)], idx_i, sem_l),
            pltpu.async_copy(item_j_hbm.at[pl.ds(base, BPW)], idx_j, sem_l),
        ]
        for cp in stage:
            cp.wait()

    with jax.named_scope("fire_rows"):
        def fire(g, _):
            gsl = pl.ds(g * L, L)
            vu = idx_u[gsl]
            vi = idx_i[gsl]
            vj = idx_j[gsl]
            for l in range(L):
                r = g * L + l
                dst = pl.ds(r, 1)
                pltpu.async_copy(eu_hbm.at[pl.ds(vu[l], 1)],
                                 u_rows.at[dst], sem_r)
                pltpu.async_copy(eit_hbm.at[pl.ds(vi[l], 1)],
                                 ei_rows.at[dst], sem_r)
                pltpu.async_copy(eit_hbm.at[pl.ds(vj[l], 1)],
                                 ej_rows.at[dst], sem_r)
            return _
        lax.fori_loop(0, NG, fire, 0)

    with jax.named_scope("fire_lin"):
        off_u = jnp.full((L,), U, jnp.int32)
        for c in range(NG):
            sl = pl.ds(c * L, L)
            idx_il[sl] = idx_i[sl] + off_u
            idx_jl[sl] = idx_j[sl] + off_u
        lin_cps = [
            pltpu.async_copy(lin_hbm.at[idx_u], lin_u, sem_l),
            pltpu.async_copy(lin_hbm.at[idx_il], lin_iv, sem_l),
            pltpu.async_copy(lin_hbm.at[idx_jl], lin_jv, sem_l),
        ]

    with jax.named_scope("row_wait"):
        pltpu.make_async_copy(eu_hbm.at[pl.ds(0, BPW)], u_rows, sem_r).wait()
        pltpu.make_async_copy(eit_hbm.at[pl.ds(0, BPW)], ei_rows, sem_r).wait()
        pltpu.make_async_copy(eit_hbm.at[pl.ds(0, BPW)], ej_rows, sem_r).wait()

    iota = lax.iota(jnp.int32, L)
    last_lane = iota == jnp.full((L,), L - 1, jnp.int32)

    with jax.named_scope("dots"):
        @plsc.parallel_loop(0, BPW, step=1, unroll=4)
        def _row_dot(r):
            acc_i = None
            acc_j = None
            for c in range(DC):
                sl = pl.ds(c * L, L)
                uv = u_rows[r, sl]
                eiv = ei_rows[r, sl]
                ejv = ej_rows[r, sl]
                if acc_i is None:
                    acc_i = uv * eiv
                    acc_j = uv * ejv
                else:
                    acc_i = acc_i + uv * eiv
                    acc_j = acc_j + uv * ejv
            ridx = lax.broadcast(r, (L,))
            plsc.store_scatter(out_i_v, [ridx], plsc.cumsum(acc_i),
                               mask=last_lane)
            plsc.store_scatter(out_j_v, [ridx], plsc.cumsum(acc_j),
                               mask=last_lane)

    with jax.named_scope("lin_part"):
        for cp in lin_cps:
            cp.wait()
        scale = jnp.full((L,), 0.3, jnp.float32)
        for g in range(NG):
            sl = pl.ds(g * L, L)
            lu = lin_u[sl]
            out_i_v[sl] = out_i_v[sl] + scale * (lu + lin_iv[sl])
            out_j_v[sl] = out_j_v[sl] + scale * (lu + lin_jv[sl])

    with jax.named_scope("out_copy"):
        outs = [
            pltpu.async_copy(out_i_v, out_i_hbm.at[pl.ds(base, BPW)], sem_l),
            pltpu.async_copy(out_j_v, out_j_hbm.at[pl.ds(base, BPW)], sem_l),
        ]
        for cp in outs:
            cp.wait()


def kernel(user, item_i, item_j, linear_w, embed_user_w, embed_item_w):
    user = user.astype(jnp.int32)
    item_i = item_i.astype(jnp.int32)
    item_j = item_j.astype(jnp.int32)
    lin_flat = linear_w.reshape(-1)
    return _fm_kernel(user, item_i, item_j, lin_flat, embed_user_w,
                      embed_item_w)


# speedup vs baseline: 1.0230x; 1.0230x over previous
import functools

import jax
import jax.numpy as jnp
from jax import lax
from jax.experimental import pallas as pl
from jax.experimental.pallas import tpu as pltpu
from jax.experimental.pallas import tpu_sc as plsc

B = 4096
U = 4096
I = 8192
D = 64

NC = 2
NS = 16
NW = NC * NS
L = 16
BPW = B // NW
NG = BPW // L
DC = D // L
NCH = 4
CH = BPW // NCH

_mesh = plsc.VectorSubcoreMesh(core_axis_name="c", subcore_axis_name="s")


@functools.partial(
    pl.kernel,
    mesh=_mesh,
    out_type=(
        jax.ShapeDtypeStruct((B,), jnp.float32),
        jax.ShapeDtypeStruct((B,), jnp.float32),
    ),
    scratch_types=dict(
        idx_u=pltpu.VMEM((BPW,), jnp.int32),
        idx_i=pltpu.VMEM((BPW,), jnp.int32),
        idx_j=pltpu.VMEM((BPW,), jnp.int32),
        idx_il=pltpu.VMEM((BPW,), jnp.int32),
        idx_jl=pltpu.VMEM((BPW,), jnp.int32),
        u_rows=pltpu.VMEM((BPW, D), jnp.float32),
        ei_rows=pltpu.VMEM((BPW, D), jnp.float32),
        ej_rows=pltpu.VMEM((BPW, D), jnp.float32),
        lin_u=pltpu.VMEM((BPW,), jnp.float32),
        lin_iv=pltpu.VMEM((BPW,), jnp.float32),
        lin_jv=pltpu.VMEM((BPW,), jnp.float32),
        out_i_v=pltpu.VMEM((BPW,), jnp.float32),
        out_j_v=pltpu.VMEM((BPW,), jnp.float32),
        sem_a=pltpu.SemaphoreType.DMA,
        sem_b=pltpu.SemaphoreType.DMA,
        sem_c=pltpu.SemaphoreType.DMA,
        sem_d=pltpu.SemaphoreType.DMA,
        sem_l=pltpu.SemaphoreType.DMA,
    ),
    compiler_params=pltpu.CompilerParams(
        needs_layout_passes=False, use_tc_tiling_on_sc=False),
)
def _fm_kernel(
    user_hbm, item_i_hbm, item_j_hbm, lin_hbm, eu_hbm, eit_hbm,
    out_i_hbm, out_j_hbm,
    *, idx_u, idx_i, idx_j, idx_il, idx_jl, u_rows, ei_rows, ej_rows,
    lin_u, lin_iv, lin_jv, out_i_v, out_j_v, sem_a, sem_b, sem_c, sem_d,
    sem_l,
):
    wid = lax.axis_index("s") * NC + lax.axis_index("c")
    base = wid * BPW

    stage = [
        pltpu.async_copy(user_hbm.at[pl.ds(base, BPW)], idx_u, sem_l),
        pltpu.async_copy(item_i_hbm.at[pl.ds(base, BPW)], idx_i, sem_l),
        pltpu.async_copy(item_j_hbm.at[pl.ds(base, BPW)], idx_j, sem_l),
    ]
    for cp in stage:
        cp.wait()

    chunk_cps = []
    for k, sem in ((0, sem_a), (1, sem_b), (2, sem_c), (3, sem_d)):
        sl = pl.ds(k * CH, CH)
        chunk_cps.append([
            pltpu.async_copy(eu_hbm.at[idx_u.at[sl]], u_rows.at[sl], sem),
            pltpu.async_copy(eit_hbm.at[idx_i.at[sl]], ei_rows.at[sl], sem),
            pltpu.async_copy(eit_hbm.at[idx_j.at[sl]], ej_rows.at[sl], sem),
        ])

    off_u = jnp.full((L,), U, jnp.int32)
    for c in range(NG):
        sl = pl.ds(c * L, L)
        idx_il[sl] = idx_i[sl] + off_u
        idx_jl[sl] = idx_j[sl] + off_u
    lin_cps = [
        pltpu.async_copy(lin_hbm.at[idx_u], lin_u, sem_l),
        pltpu.async_copy(lin_hbm.at[idx_il], lin_iv, sem_l),
        pltpu.async_copy(lin_hbm.at[idx_jl], lin_jv, sem_l),
    ]

    iota = lax.iota(jnp.int32, L)
    last_lane = iota == jnp.full((L,), L - 1, jnp.int32)

    for k in range(NCH):
        for cp in chunk_cps[k]:
            cp.wait()

        @plsc.parallel_loop(k * CH, (k + 1) * CH, step=1, unroll=4)
        def _row_dot(r):
            acc_i = None
            acc_j = None
            for c in range(DC):
                sl = pl.ds(c * L, L)
                uv = u_rows[r, sl]
                eiv = ei_rows[r, sl]
                ejv = ej_rows[r, sl]
                if acc_i is None:
                    acc_i = uv * eiv
                    acc_j = uv * ejv
                else:
                    acc_i = acc_i + uv * eiv
                    acc_j = acc_j + uv * ejv
            ridx = lax.broadcast(r, (L,))
            plsc.store_scatter(out_i_v, [ridx], plsc.cumsum(acc_i),
                               mask=last_lane)
            plsc.store_scatter(out_j_v, [ridx], plsc.cumsum(acc_j),
                               mask=last_lane)

    for cp in lin_cps:
        cp.wait()
    scale = jnp.full((L,), 0.3, jnp.float32)
    for g in range(NG):
        sl = pl.ds(g * L, L)
        lu = lin_u[sl]
        out_i_v[sl] = out_i_v[sl] + scale * (lu + lin_iv[sl])
        out_j_v[sl] = out_j_v[sl] + scale * (lu + lin_jv[sl])

    outs = [
        pltpu.async_copy(out_i_v, out_i_hbm.at[pl.ds(base, BPW)], sem_l),
        pltpu.async_copy(out_j_v, out_j_hbm.at[pl.ds(base, BPW)], sem_l),
    ]
    for cp in outs:
        cp.wait()


def kernel(user, item_i, item_j, linear_w, embed_user_w, embed_item_w):
    user = user.astype(jnp.int32)
    item_i = item_i.astype(jnp.int32)
    item_j = item_j.astype(jnp.int32)
    lin_flat = linear_w.reshape(-1)
    return _fm_kernel(user, item_i, item_j, lin_flat, embed_user_w,
                      embed_item_w)
